# Initial kernel scaffold; baseline (speedup 1.0000x reference)
import jax
import jax.numpy as jnp
from jax.experimental import pallas as pl


def kernel(input_0, input_1, input_2, input_3, rois, rois_counts, level):
    raise NotImplementedError("write your pallas kernel here")


# speedup vs baseline: 4.1584x; 4.1584x over previous
import functools

import jax
import jax.numpy as jnp
import numpy as np
from jax.experimental import pallas as pl
from jax.experimental.pallas import tpu as pltpu

_N = 1000
_C = 256
_HS = (200, 100, 50, 25)
_SCALES = (0.25, 0.125, 0.0625, 0.03125)
_OUT = 7
_SR = 2
_P = _OUT * _SR
_OFFSETS = (0, 40000, 50000, 52500)
_S = 53125
_S_PAD = 53128


def _roi_kernel(rb_ref, xx_ref, wy_ref, wx_ref, table_ref, out_ref):
    n = pl.program_id(0)
    base = n * (2 * _P)

    def bin_body(b, _):
        oh = b // _OUT
        ow = b % _OUT
        acc = jnp.zeros((1, _C), jnp.float32)
        for i in (0, 1):
            for j in (0, 1):
                py = base + 2 * oh + i
                px = base + 2 * ow + j
                r0 = rb_ref[py]
                r1 = rb_ref[_P + py]
                c0 = xx_ref[px]
                c1 = xx_ref[_P + px]
                wy0 = wy_ref[py]
                wy1 = wy_ref[_P + py]
                wx0 = wx_ref[px]
                wx1 = wx_ref[_P + px]
                acc = acc + (wy0 * wx0) * table_ref[pl.ds(r0 + c0, 1), :]
                acc = acc + (wy0 * wx1) * table_ref[pl.ds(r0 + c1, 1), :]
                acc = acc + (wy1 * wx0) * table_ref[pl.ds(r1 + c0, 1), :]
                acc = acc + (wy1 * wx1) * table_ref[pl.ds(r1 + c1, 1), :]
        out_ref[0, pl.ds(b, 1), :] = acc * 0.25
        return 0

    jax.lax.fori_loop(0, _OUT * _OUT, bin_body, 0)


def _build_table(feats):
    rows = [f[0].transpose(1, 2, 0).reshape(h * h, _C)
            for f, h in zip(feats, _HS)]
    rows.append(jnp.zeros((_S_PAD - _S, _C), jnp.float32))
    return jnp.concatenate(rows, axis=0)


def _indices_weights(rois, level):
    lvl = level.astype(jnp.int32)
    scales = jnp.array(_SCALES, jnp.float32)
    sizes = jnp.array(_HS, jnp.float32)
    offs = jnp.array(_OFFSETS, jnp.int32)
    wl_i = jnp.array(_HS, jnp.int32)
    sc = scales[lvl]
    hl = sizes[lvl]
    x1 = rois[:, 1] * sc
    y1 = rois[:, 2] * sc
    x2 = rois[:, 3] * sc
    y2 = rois[:, 4] * sc
    roi_w = jnp.maximum(x2 - x1, 1.0)
    roi_h = jnp.maximum(y2 - y1, 1.0)
    bin_h = roi_h / _OUT
    bin_w = roi_w / _OUT
    off = (jnp.arange(_P, dtype=jnp.float32) + 0.5) / _SR
    ys = jnp.clip(y1[:, None] + off[None, :] * bin_h[:, None], 0.0,
                  hl[:, None] - 1.0)
    xs = jnp.clip(x1[:, None] + off[None, :] * bin_w[:, None], 0.0,
                  hl[:, None] - 1.0)
    y0f = jnp.floor(ys)
    x0f = jnp.floor(xs)
    ly = ys - y0f
    lx = xs - x0f
    y0 = y0f.astype(jnp.int32)
    x0 = x0f.astype(jnp.int32)
    hi = (hl[:, None] - 1.0).astype(jnp.int32)
    y1i = jnp.minimum(y0 + 1, hi)
    x1i = jnp.minimum(x0 + 1, hi)
    base = offs[lvl][:, None]
    w = wl_i[lvl][:, None]
    rb = jnp.stack([base + y0 * w, base + y1i * w], axis=1)
    xx = jnp.stack([x0, x1i], axis=1)
    wy = jnp.stack([1.0 - ly, ly], axis=1)
    wx = jnp.stack([1.0 - lx, lx], axis=1)
    return (rb.reshape(-1), xx.reshape(-1), wy.reshape(-1), wx.reshape(-1))


@functools.partial(jax.jit, static_argnames=("interpret",))
def _roi_align(input_0, input_1, input_2, input_3, rois, level,
               interpret=False):
    table = _build_table((input_0, input_1, input_2, input_3))
    rb, xx, wy, wx = _indices_weights(rois, level)
    n = rois.shape[0]
    grid_spec = pltpu.PrefetchScalarGridSpec(
        num_scalar_prefetch=4,
        grid=(n,),
        in_specs=[
            pl.BlockSpec((_S_PAD, _C), lambda i, *_: (0, 0)),
        ],
        out_specs=pl.BlockSpec((1, _OUT * _OUT, _C), lambda i, *_: (i, 0, 0)),
    )
    out = pl.pallas_call(
        _roi_kernel,
        grid_spec=grid_spec,
        out_shape=jax.ShapeDtypeStruct((n, _OUT * _OUT, _C), jnp.float32),
        interpret=interpret,
    )(rb, xx, wy, wx, table)
    out = out.reshape(n, _OUT, _OUT, _C)
    return jnp.transpose(out, (0, 3, 1, 2))


def kernel(input_0, input_1, input_2, input_3, rois, rois_counts, level):
    del rois_counts
    return _roi_align(input_0, input_1, input_2, input_3, rois, level)
